# Initial kernel scaffold; baseline (speedup 1.0000x reference)
import jax
import jax.numpy as jnp
from jax.experimental import pallas as pl


def kernel(z, W):
    raise NotImplementedError("write your pallas kernel here")


# speedup vs baseline: 1.0330x; 1.0330x over previous
import functools

import jax
import jax.numpy as jnp
from jax import lax
from jax.experimental import pallas as pl
from jax.experimental.pallas import tpu as pltpu

BETA = 0.25
NE = 8192
DE = 256

BM = 1024
BN = 1024


def _argmin_body(z2_ref, w2_ref, z_ref, w_ref, idx_ref):
    zb = z_ref[...]
    z2 = z2_ref[...]

    def body(j, carry):
        run_min, run_arg = carry
        wb = w_ref[pl.ds(j * BN, BN), :]
        w2 = w2_ref[:, pl.ds(j * BN, BN)]
        m = lax.dot_general(zb, wb, (((1,), (1,)), ((), ())),
                            preferred_element_type=jnp.float32)
        d = (z2 + w2) - 2.0 * m
        bmin = jnp.min(d, axis=1, keepdims=True)
        iota = lax.broadcasted_iota(jnp.int32, (BM, BN), 1)
        barg = (jnp.min(jnp.where(d == bmin, iota, NE), axis=1, keepdims=True)
                + j * BN)
        better = bmin < run_min
        return (jnp.where(better, bmin, run_min),
                jnp.where(better, barg, run_arg))

    init = (jnp.full((BM, 1), jnp.inf, jnp.float32),
            jnp.zeros((BM, 1), jnp.int32))
    _, run_arg = lax.fori_loop(0, NE // BN, body, init)
    idx_ref[...] = run_arg


def _compute_indices(z_flat, W, z2, w2):
    grid = (z_flat.shape[0] // BM,)
    idx2d = pl.pallas_call(
        _argmin_body,
        grid=grid,
        in_specs=[
            pl.BlockSpec((BM, 1), lambda i: (i, 0)),
            pl.BlockSpec((1, NE), lambda i: (0, 0)),
            pl.BlockSpec((BM, DE), lambda i: (i, 0)),
            pl.BlockSpec((NE, DE), lambda i: (0, 0)),
        ],
        out_specs=pl.BlockSpec((BM, 1), lambda i: (i, 0)),
        out_shape=jax.ShapeDtypeStruct((z_flat.shape[0], 1), jnp.int32),
    )(z2, w2, z_flat, W)
    return idx2d.reshape(-1)


def kernel(z, W):
    z_p = jnp.transpose(z, (0, 2, 3, 1))
    z_flat = z_p.reshape(-1, DE)
    z2 = jnp.sum(z_flat ** 2, axis=1, keepdims=True)
    w2 = jnp.sum(W ** 2, axis=1).reshape(1, NE)

    encoding_indices = _compute_indices(z_flat, W, z2, w2)

    encodings = jax.nn.one_hot(encoding_indices, NE, dtype=z_p.dtype)
    quantized = W[encoding_indices]
    s = jnp.mean((quantized - z_flat) ** 2)
    loss = s + BETA * s
    quantized_st = z_flat + (quantized - z_flat)
    probs = jnp.mean(encodings, axis=0)
    perplexity = jnp.exp(-jnp.sum(probs * jnp.log(probs + 1e-10)))
    quantized_out = jnp.transpose(quantized_st.reshape(z_p.shape), (0, 3, 1, 2))
    return (loss, quantized_out, perplexity, encodings, encoding_indices)
